# Initial kernel scaffold; baseline (speedup 1.0000x reference)
import jax
import jax.numpy as jnp
from jax.experimental import pallas as pl


def kernel(rel, arg1, fact_rel, fact_arg1, fact_arg2):
    raise NotImplementedError("write your pallas kernel here")


# speedup vs baseline: 2.0396x; 2.0396x over previous
import functools

import jax
import jax.numpy as jnp
from jax.experimental import pallas as pl
from jax.experimental.pallas import tpu as pltpu
from jax.experimental.pallas import tpu_sc as plsc

B = 1024
F = 100000
D = 64
QD = 2 * D
K = 5
NBLK = 50
BLK = F // NBLK
RW = 16


def _topk_body(q_ref, q2_ref, k_ref, k2_ref, out_v_ref, out_i_ref,
               rv_ref, ri_ref):
    blk = pl.program_id(0)

    @pl.when(blk == 0)
    def _init():
        rv_ref[...] = jnp.full((B, RW), jnp.inf, jnp.float32)
        ri_ref[...] = jnp.zeros((B, RW), jnp.int32)

    q = q_ref[...]
    kb = k_ref[...]
    qk = jax.lax.dot_general(q, kb, (((1,), (1,)), ((), ())),
                             preferred_element_type=jnp.float32)
    d2 = (q2_ref[...] - 2.0 * qk) + k2_ref[0]

    rv = rv_ref[...]
    ri = ri_ref[...]
    itb = jax.lax.broadcasted_iota(jnp.int32, (B, BLK), 1)
    it16 = jax.lax.broadcasted_iota(jnp.int32, (B, RW), 1)
    BIG = jnp.int32(2 ** 30)
    INF = jnp.float32(jnp.inf)
    base = blk * BLK

    vs, gs = [], []
    for _ in range(K):
        vb = jnp.min(d2, axis=1, keepdims=True)
        cb = jnp.min(jnp.where(d2 == vb, itb, BIG), axis=1, keepdims=True)
        vr = jnp.min(rv, axis=1, keepdims=True)
        cr = jnp.min(jnp.where(rv == vr, it16, BIG), axis=1, keepdims=True)
        gr = jnp.min(jnp.where(it16 == cr, ri, BIG), axis=1, keepdims=True)
        take_run = vr <= vb
        v = jnp.where(take_run, vr, vb)
        g = jnp.where(take_run, gr, base + cb)
        vs.append(v)
        gs.append(g)
        rv = jnp.where((it16 == cr) & take_run, INF, rv)
        d2 = jnp.where((itb == cb) & (~take_run), INF, d2)

    rv_new = jnp.concatenate(
        vs + [jnp.full((B, RW - K), jnp.inf, jnp.float32)], axis=1)
    ri_new = jnp.concatenate(
        gs + [jnp.zeros((B, RW - K), jnp.int32)], axis=1)
    rv_ref[...] = rv_new
    ri_ref[...] = ri_new

    @pl.when(blk == NBLK - 1)
    def _finish():
        out_v_ref[...] = jnp.exp(rv_new * -0.5)
        out_i_ref[...] = ri_new


def _topk_call(batch_q, q2, keys, k2):
    return pl.pallas_call(
        _topk_body,
        grid=(NBLK,),
        in_specs=[
            pl.BlockSpec((B, QD), lambda i: (0, 0)),
            pl.BlockSpec((B, 1), lambda i: (0, 0)),
            pl.BlockSpec((BLK, QD), lambda i: (i, 0)),
            pl.BlockSpec((1, 1, BLK), lambda i: (i, 0, 0)),
        ],
        out_specs=[
            pl.BlockSpec((B, RW), lambda i: (0, 0)),
            pl.BlockSpec((B, RW), lambda i: (0, 0)),
        ],
        out_shape=[
            jax.ShapeDtypeStruct((B, RW), jnp.float32),
            jax.ShapeDtypeStruct((B, RW), jnp.int32),
        ],
        scratch_shapes=[
            pltpu.VMEM((B, RW), jnp.float32),
            pltpu.VMEM((B, RW), jnp.int32),
        ],
    )(batch_q, q2, keys, k2)


NW = 32
BPW = (B * K) // NW
GW = 128


@functools.lru_cache(maxsize=None)
def _make_sc_gather():
    mesh = plsc.VectorSubcoreMesh(core_axis_name="c", subcore_axis_name="s")

    @functools.partial(
        pl.kernel, mesh=mesh,
        out_type=jax.ShapeDtypeStruct((B * K, GW), jnp.float32),
        scratch_types=[
            pltpu.VMEM((BPW,), jnp.int32),
            pltpu.VMEM((BPW, GW), jnp.float32),
            pltpu.SemaphoreType.DMA,
        ],
    )
    def _gather(idx_hbm, table_hbm, out_hbm, idx_v, rows_v, sem):
        wid = jax.lax.axis_index("s") * 2 + jax.lax.axis_index("c")
        base = wid * BPW
        pltpu.sync_copy(idx_hbm.at[pl.ds(base, BPW)], idx_v)
        pltpu.async_copy(table_hbm.at[idx_v], rows_v, sem).wait()
        pltpu.sync_copy(rows_v, out_hbm.at[pl.ds(base, BPW)])

    return _gather


def kernel(rel, arg1, fact_rel, fact_arg1, fact_arg2):
    batch_q = jnp.concatenate([rel, arg1], axis=1)
    keys = jnp.concatenate([fact_rel, fact_arg1], axis=1)
    q2 = jnp.sum(batch_q ** 2, axis=1, keepdims=True)
    k2 = jnp.sum(keys ** 2, axis=1).reshape(NBLK, 1, BLK)
    scores16, idx16 = _topk_call(batch_q, q2, keys, k2)
    scores = scores16[:, :K]
    idx = idx16[:, :K]
    table = jnp.concatenate([fact_arg2, fact_arg2], axis=1)
    subs = _make_sc_gather()(idx.reshape(-1), table)[:, :D].reshape(B, K, D)
    return scores, subs
